# baseline (speedup 1.0000x reference)
import functools

import numpy as np
import jax
import jax.numpy as jnp
from jax import lax
from jax.experimental import pallas as pl
from jax.experimental.pallas import tpu as pltpu
from jax.experimental.pallas import tpu_sc as plsc

_N_EDGES = 3200000
_DROP_P = 0.2
_C = 128
_NW = 32


def _kept_indices() -> np.ndarray:
    key = jax.random.fold_in(jax.random.key(0), 12345)
    keep = np.asarray(jax.random.uniform(key, (_N_EDGES,)) >= _DROP_P)
    return np.nonzero(keep)[0].astype(np.int32)


_IDX = _kept_indices()
_K = int(_IDX.size)
_NCH = _K // _C
_T = _K - _NCH * _C
_TAIL_OFF = _NCH * _C
_CPW = -(-_NCH // _NW)

_IDX_ARR = jnp.asarray(_IDX)

_mesh = plsc.VectorSubcoreMesh(core_axis_name="c", subcore_axis_name="s")


@functools.partial(
    pl.kernel,
    out_type=(
        jax.ShapeDtypeStruct((2, _K), jnp.int32),
        jax.ShapeDtypeStruct((_K, 16), jnp.float32),
    ),
    mesh=_mesh,
    scratch_types=[
        pltpu.VMEM((_C,), jnp.int32),
        pltpu.VMEM((_C,), jnp.int32),
        pltpu.VMEM((_C,), jnp.int32),
        pltpu.VMEM((_C, 16), jnp.float32),
        pltpu.SemaphoreType.DMA,
        pltpu.SemaphoreType.DMA,
        pltpu.SemaphoreType.DMA,
    ],
    compiler_params=pltpu.CompilerParams(use_tc_tiling_on_sc=False),
)
def _sc_compact(idx_hbm, eidx_hbm, attr_hbm, out_eidx_hbm, out_attr_hbm,
                idx_v, r0_v, r1_v, rows_v, sem_a, sem_b, sem_c):
    wid = lax.axis_index("s") * 2 + lax.axis_index("c")
    c0 = wid * _CPW
    c1 = jnp.minimum(c0 + _CPW, _NCH)

    def body(c, carry):
        off = pl.multiple_of(c * _C, _C)
        pltpu.sync_copy(idx_hbm.at[pl.ds(off, _C)], idx_v)
        ca = pltpu.async_copy(attr_hbm.at[idx_v], rows_v, sem_a)
        cb = pltpu.async_copy(eidx_hbm.at[0].at[idx_v], r0_v, sem_b)
        cc = pltpu.async_copy(eidx_hbm.at[1].at[idx_v], r1_v, sem_c)
        ca.wait()
        cb.wait()
        cc.wait()
        pltpu.sync_copy(rows_v, out_attr_hbm.at[pl.ds(off, _C)])
        pltpu.sync_copy(r0_v, out_eidx_hbm.at[0].at[pl.ds(off, _C)])
        pltpu.sync_copy(r1_v, out_eidx_hbm.at[1].at[pl.ds(off, _C)])
        return carry

    lax.fori_loop(c0, c1, body, 0)

    @pl.when(wid == _NW - 1)
    def _tail():
        idx_t = idx_v.at[pl.ds(0, _T)]
        r0_t = r0_v.at[pl.ds(0, _T)]
        r1_t = r1_v.at[pl.ds(0, _T)]
        rows_t = rows_v.at[pl.ds(0, _T)]
        pltpu.sync_copy(idx_hbm.at[pl.ds(_TAIL_OFF, _T)], idx_t)
        ca = pltpu.async_copy(attr_hbm.at[idx_t], rows_t, sem_a)
        cb = pltpu.async_copy(eidx_hbm.at[0].at[idx_t], r0_t, sem_b)
        cc = pltpu.async_copy(eidx_hbm.at[1].at[idx_t], r1_t, sem_c)
        ca.wait()
        cb.wait()
        cc.wait()
        pltpu.sync_copy(rows_t, out_attr_hbm.at[pl.ds(_TAIL_OFF, _T)])
        pltpu.sync_copy(r0_t, out_eidx_hbm.at[0].at[pl.ds(_TAIL_OFF, _T)])
        pltpu.sync_copy(r1_t, out_eidx_hbm.at[1].at[pl.ds(_TAIL_OFF, _T)])


def kernel(edge_index, edge_attr):
    return _sc_compact(_IDX_ARR, edge_index, edge_attr)


# speedup vs baseline: 1.2103x; 1.2103x over previous
import functools

import numpy as np
import jax
import jax.numpy as jnp
from jax import lax
from jax.experimental import pallas as pl
from jax.experimental.pallas import tpu as pltpu
from jax.experimental.pallas import tpu_sc as plsc

_N_EDGES = 3200000
_DROP_P = 0.2
_C = 128
_S = 4
_B = _S * _C
_NW = 32


def _kept_indices() -> np.ndarray:
    key = jax.random.fold_in(jax.random.key(0), 12345)
    keep = np.asarray(jax.random.uniform(key, (_N_EDGES,)) >= _DROP_P)
    return np.nonzero(keep)[0].astype(np.int32)


_IDX = _kept_indices()
_K = int(_IDX.size)
_NCH = _K // _C
_T = _K - _NCH * _C
_TAIL_OFF = _NCH * _C
_NSC = _NCH // _S
_SPW = -(-_NSC // _NW)


def _padded_idx() -> np.ndarray:
    pad = np.zeros(((_NCH + 1) * _C,), np.int32)
    pad[:_K] = _IDX
    return pad.reshape(_NCH + 1, _C)


_IDX_ARR = jnp.asarray(_padded_idx())

_mesh = plsc.VectorSubcoreMesh(core_axis_name="c", subcore_axis_name="s")


@functools.partial(
    pl.kernel,
    out_type=(
        jax.ShapeDtypeStruct((2, _K), jnp.int32),
        jax.ShapeDtypeStruct((_K, 16), jnp.float32),
    ),
    mesh=_mesh,
    scratch_types=[
        [pltpu.VMEM((_S, _C), jnp.int32)] * 2,
        [pltpu.VMEM((_B,), jnp.int32)] * 2,
        [pltpu.VMEM((_B,), jnp.int32)] * 2,
        [pltpu.VMEM((_B, 16), jnp.float32)] * 2,
        [pltpu.SemaphoreType.DMA] * 2,
        [pltpu.SemaphoreType.DMA] * 2,
        [pltpu.SemaphoreType.DMA] * 2,
    ],
    compiler_params=pltpu.CompilerParams(use_tc_tiling_on_sc=False),
)
def _sc_compact(idx_hbm, eidx_hbm, attr_hbm, out_eidx_hbm, out_attr_hbm,
                idx_v, r0_v, r1_v, rows_v, sem_i, sem_g, sem_w):
    wid = lax.axis_index("s") * 2 + lax.axis_index("c")
    s0 = wid * _SPW
    s1 = jnp.minimum(s0 + _SPW, _NSC)
    my_n = s1 - s0

    def stage(g, b):
        return pltpu.async_copy(
            idx_hbm.at[pl.ds(pl.multiple_of(g * _S, _S), _S)], idx_v[b],
            sem_i[b])

    def fire_gathers(g, b):
        cps = []
        for j in range(_S):
            ij = idx_v[b].at[j]
            cps.append(pltpu.async_copy(
                attr_hbm.at[ij], rows_v[b].at[pl.ds(j * _C, _C)], sem_g[b]))
            cps.append(pltpu.async_copy(
                eidx_hbm.at[0].at[ij], r0_v[b].at[pl.ds(j * _C, _C)],
                sem_g[b]))
            cps.append(pltpu.async_copy(
                eidx_hbm.at[1].at[ij], r1_v[b].at[pl.ds(j * _C, _C)],
                sem_g[b]))
        return cps

    def fire_writes(g, b):
        off = pl.multiple_of(g * _B, _B)
        return [
            pltpu.async_copy(rows_v[b], out_attr_hbm.at[pl.ds(off, _B)],
                             sem_w[b]),
            pltpu.async_copy(r0_v[b], out_eidx_hbm.at[0].at[pl.ds(off, _B)],
                             sem_w[b]),
            pltpu.async_copy(r1_v[b], out_eidx_hbm.at[1].at[pl.ds(off, _B)],
                             sem_w[b]),
        ]

    def wait_all(cps):
        for cp in cps:
            cp.wait()

    @pl.when(my_n > 0)
    def _p0():
        stage(s0, 0)

    @pl.when(my_n > 1)
    def _p1():
        stage(s0 + 1, 1)

    def pipe_body(k, carry):
        g0 = s0 + 2 * k
        g1 = g0 + 1

        @pl.when(g0 < s1)
        def _b0():
            pltpu.make_async_copy(
                idx_hbm.at[pl.ds(0, _S)], idx_v[0], sem_i[0]).wait()
            cps_g = fire_gathers(g0, 0)
            wait_all(cps_g)
            @pl.when(g0 + 2 < s1)
            def _():
                stage(g0 + 2, 0)
            cps_w = fire_writes(g0, 0)

            @pl.when(g1 < s1)
            def _b1():
                pltpu.make_async_copy(
                    idx_hbm.at[pl.ds(0, _S)], idx_v[1], sem_i[1]).wait()
                cps_g1 = fire_gathers(g1, 1)
                wait_all(cps_g1)
                @pl.when(g1 + 2 < s1)
                def _():
                    stage(g1 + 2, 1)
                wait_all(fire_writes(g1, 1))

            wait_all(cps_w)

        return carry

    lax.fori_loop(0, (_SPW + 1) // 2, pipe_body, 0)

    @pl.when(wid == _NW - 1)
    def _tail():
        idx_t = idx_v[0].at[0].at[pl.ds(0, _T)]
        r0_t = r0_v[0].at[pl.ds(0, _T)]
        r1_t = r1_v[0].at[pl.ds(0, _T)]
        rows_t = rows_v[0].at[pl.ds(0, _T)]
        pltpu.sync_copy(idx_hbm.at[_NCH].at[pl.ds(0, _T)], idx_t)
        ca = pltpu.async_copy(attr_hbm.at[idx_t], rows_t, sem_g[0])
        cb = pltpu.async_copy(eidx_hbm.at[0].at[idx_t], r0_t, sem_g[0])
        cc = pltpu.async_copy(eidx_hbm.at[1].at[idx_t], r1_t, sem_g[0])
        ca.wait()
        cb.wait()
        cc.wait()
        pltpu.sync_copy(rows_t, out_attr_hbm.at[pl.ds(_TAIL_OFF, _T)])
        pltpu.sync_copy(r0_t, out_eidx_hbm.at[0].at[pl.ds(_TAIL_OFF, _T)])
        pltpu.sync_copy(r1_t, out_eidx_hbm.at[1].at[pl.ds(_TAIL_OFF, _T)])


def kernel(edge_index, edge_attr):
    return _sc_compact(_IDX_ARR, edge_index, edge_attr)


# speedup vs baseline: 1.2405x; 1.0250x over previous
import functools

import numpy as np
import jax
import jax.numpy as jnp
from jax import lax
from jax.experimental import pallas as pl
from jax.experimental.pallas import tpu as pltpu
from jax.experimental.pallas import tpu_sc as plsc

_N_EDGES = 3200000
_DROP_P = 0.2
_C = 128
_S = 4
_B = _S * _C
_NW = 32


def _kept_indices() -> np.ndarray:
    key = jax.random.fold_in(jax.random.key(0), 12345)
    keep = np.asarray(jax.random.uniform(key, (_N_EDGES,)) >= _DROP_P)
    return np.nonzero(keep)[0].astype(np.int32)


_IDX = _kept_indices()
_K = int(_IDX.size)
_NCH = _K // _C
_T = _K - _NCH * _C
_TAIL_OFF = _NCH * _C
_NSC = _NCH // _S
_SPW = -(-_NSC // _NW)


def _padded_idx() -> np.ndarray:
    pad = np.zeros(((_NCH + 1) * _C,), np.int32)
    pad[:_K] = _IDX
    return pad


_IDX_ARR = jnp.asarray(_padded_idx())

_mesh = plsc.VectorSubcoreMesh(core_axis_name="c", subcore_axis_name="s")


@functools.partial(
    pl.kernel,
    out_type=(
        jax.ShapeDtypeStruct((_K,), jnp.int32),
        jax.ShapeDtypeStruct((_K,), jnp.int32),
        jax.ShapeDtypeStruct((_K, 16), jnp.float32),
    ),
    mesh=_mesh,
    scratch_types=[
        [pltpu.VMEM((_B,), jnp.int32)] * 2,
        [pltpu.VMEM((_B,), jnp.int32)] * 2,
        [pltpu.VMEM((_B,), jnp.int32)] * 2,
        [pltpu.VMEM((_B, 16), jnp.float32)] * 2,
        [pltpu.SemaphoreType.DMA] * 2,
        [pltpu.SemaphoreType.DMA] * 2,
        [pltpu.SemaphoreType.DMA] * 2,
    ],
    compiler_params=pltpu.CompilerParams(use_tc_tiling_on_sc=False),
)
def _sc_compact(idx_hbm, e0_hbm, e1_hbm, attr_hbm,
                out_e0_hbm, out_e1_hbm, out_attr_hbm,
                idx_v, r0_v, r1_v, rows_v, sem_i, sem_g, sem_w):
    wid = lax.axis_index("s") * 2 + lax.axis_index("c")
    s0 = wid * _SPW
    s1 = jnp.minimum(s0 + _SPW, _NSC)
    my_n = s1 - s0

    def stage(g, b):
        return pltpu.async_copy(
            idx_hbm.at[pl.ds(pl.multiple_of(g * _B, _B), _B)], idx_v[b],
            sem_i[b])

    def fire_gathers(g, b):
        cps = []
        for j in range(_S):
            ij = idx_v[b].at[pl.ds(j * _C, _C)]
            sl = pl.ds(j * _C, _C)
            cps.append(pltpu.async_copy(
                attr_hbm.at[ij], rows_v[b].at[sl], sem_g[b]))
            cps.append(pltpu.async_copy(e0_hbm.at[ij], r0_v[b].at[sl],
                                        sem_g[b]))
            cps.append(pltpu.async_copy(e1_hbm.at[ij], r1_v[b].at[sl],
                                        sem_g[b]))
        return cps

    def fire_writes(g, b):
        off = pl.multiple_of(g * _B, _B)
        return [
            pltpu.async_copy(rows_v[b], out_attr_hbm.at[pl.ds(off, _B)],
                             sem_w[b]),
            pltpu.async_copy(r0_v[b], out_e0_hbm.at[pl.ds(off, _B)],
                             sem_w[b]),
            pltpu.async_copy(r1_v[b], out_e1_hbm.at[pl.ds(off, _B)],
                             sem_w[b]),
        ]

    def wait_all(cps):
        for cp in cps:
            cp.wait()

    @pl.when(my_n > 0)
    def _p0():
        stage(s0, 0)

    @pl.when(my_n > 1)
    def _p1():
        stage(s0 + 1, 1)

    def pipe_body(k, carry):
        g0 = s0 + 2 * k
        g1 = g0 + 1

        @pl.when(g0 < s1)
        def _b0():
            pltpu.make_async_copy(
                idx_hbm.at[pl.ds(0, _B)], idx_v[0], sem_i[0]).wait()
            cps_g = fire_gathers(g0, 0)
            wait_all(cps_g)
            @pl.when(g0 + 2 < s1)
            def _():
                stage(g0 + 2, 0)
            cps_w = fire_writes(g0, 0)

            @pl.when(g1 < s1)
            def _b1():
                pltpu.make_async_copy(
                    idx_hbm.at[pl.ds(0, _B)], idx_v[1], sem_i[1]).wait()
                cps_g1 = fire_gathers(g1, 1)
                wait_all(cps_g1)
                @pl.when(g1 + 2 < s1)
                def _():
                    stage(g1 + 2, 1)
                wait_all(fire_writes(g1, 1))

            wait_all(cps_w)

        return carry

    lax.fori_loop(0, (_SPW + 1) // 2, pipe_body, 0)

    @pl.when(wid == _NW - 1)
    def _tail():
        idx_t = idx_v[0].at[pl.ds(0, _T)]
        r0_t = r0_v[0].at[pl.ds(0, _T)]
        r1_t = r1_v[0].at[pl.ds(0, _T)]
        rows_t = rows_v[0].at[pl.ds(0, _T)]
        pltpu.sync_copy(idx_hbm.at[pl.ds(_TAIL_OFF, _T)], idx_t)
        ca = pltpu.async_copy(attr_hbm.at[idx_t], rows_t, sem_g[0])
        cb = pltpu.async_copy(e0_hbm.at[idx_t], r0_t, sem_g[0])
        cc = pltpu.async_copy(e1_hbm.at[idx_t], r1_t, sem_g[0])
        ca.wait()
        cb.wait()
        cc.wait()
        pltpu.sync_copy(rows_t, out_attr_hbm.at[pl.ds(_TAIL_OFF, _T)])
        pltpu.sync_copy(r0_t, out_e0_hbm.at[pl.ds(_TAIL_OFF, _T)])
        pltpu.sync_copy(r1_t, out_e1_hbm.at[pl.ds(_TAIL_OFF, _T)])


def kernel(edge_index, edge_attr):
    out_r0, out_r1, out_attr = _sc_compact(
        _IDX_ARR, edge_index[0], edge_index[1], edge_attr)
    return jnp.stack([out_r0, out_r1]), out_attr
